# baseline (device time: 13656 ns/iter reference)
import jax
import jax.numpy as jnp
from jax import lax
from jax.experimental import pallas as pl
from jax.experimental.pallas import tpu as pltpu

NCHUNK = 4
QSCALE = 12.0 / 127.0


def kernel(dy, W):
    m, k = dy.shape
    d = W.shape[0]
    rows = m // NCHUNK

    def body(dy_ref, w_ref, out_ref, w_bf_ref, p_ref, send_buf, recv_buf,
             send_sems, recv_sems, out_sems):
        my_x = lax.axis_index("x")
        my_y = lax.axis_index("y")
        my_z = lax.axis_index("z")
        nbr = (my_x, my_y, 1 - my_z)

        barrier = pltpu.get_barrier_semaphore()
        pl.semaphore_signal(
            barrier, inc=1, device_id=nbr,
            device_id_type=pl.DeviceIdType.MESH,
        )

        w_bf_ref[...] = w_ref[...].astype(jnp.bfloat16)

        rdmas = []
        for c in range(NCHUNK):
            a = dy_ref[pl.ds(c * rows, rows), :].astype(jnp.bfloat16)
            p = lax.dot_general(
                a, w_bf_ref[...], (((1,), (1,)), ((), ())),
                preferred_element_type=jnp.float32,
            )
            p_ref[c] = p
            q = jnp.clip(jnp.round(p * (1.0 / QSCALE)), -127.0, 127.0)
            send_buf[c] = q.astype(jnp.int8)
            if c == 0:
                pl.semaphore_wait(barrier, 1)
            rdma = pltpu.make_async_remote_copy(
                src_ref=send_buf.at[c],
                dst_ref=recv_buf.at[c],
                send_sem=send_sems.at[c],
                recv_sem=recv_sems.at[c],
                device_id=nbr,
                device_id_type=pl.DeviceIdType.MESH,
            )
            rdma.start()
            rdmas.append(rdma)

        out_cps = []
        for c in range(NCHUNK):
            rdmas[c].wait_recv()
            p_ref[c] = p_ref[c] + recv_buf[c].astype(jnp.float32) * QSCALE
            cp = pltpu.make_async_copy(
                p_ref.at[c],
                out_ref.at[pl.ds(c * rows, rows), :],
                out_sems.at[c],
            )
            cp.start()
            out_cps.append(cp)

        for c in range(NCHUNK):
            rdmas[c].wait_send()
            out_cps[c].wait()

    return pl.pallas_call(
        body,
        out_shape=jax.ShapeDtypeStruct((m, d), jnp.float32),
        in_specs=[
            pl.BlockSpec(memory_space=pltpu.VMEM),
            pl.BlockSpec(memory_space=pltpu.VMEM),
        ],
        out_specs=pl.BlockSpec(memory_space=pltpu.MemorySpace.HBM),
        scratch_shapes=[
            pltpu.VMEM((d, k), jnp.bfloat16),
            pltpu.VMEM((NCHUNK, rows, d), jnp.float32),
            pltpu.VMEM((NCHUNK, rows, d), jnp.int8),
            pltpu.VMEM((NCHUNK, rows, d), jnp.int8),
            pltpu.SemaphoreType.DMA((NCHUNK,)),
            pltpu.SemaphoreType.DMA((NCHUNK,)),
            pltpu.SemaphoreType.DMA((NCHUNK,)),
        ],
        compiler_params=pltpu.CompilerParams(collective_id=0),
    )(dy, W)


# device time: 13595 ns/iter; 1.0045x vs baseline; 1.0045x over previous
import jax
import jax.numpy as jnp
from jax import lax
from jax.experimental import pallas as pl
from jax.experimental.pallas import tpu as pltpu

NCHUNK = 4
QSCALE = 12.0 / 127.0


def kernel(dy, W):
    m, k = dy.shape
    d = W.shape[0]
    rows = m // NCHUNK

    def body(dy_ref, w_ref, out_ref, w_bf_ref, p_ref, send_buf, recv_buf,
             send_sems, recv_sems):
        my_x = lax.axis_index("x")
        my_y = lax.axis_index("y")
        my_z = lax.axis_index("z")
        nbr = (my_x, my_y, 1 - my_z)

        barrier = pltpu.get_barrier_semaphore()
        pl.semaphore_signal(
            barrier, inc=1, device_id=nbr,
            device_id_type=pl.DeviceIdType.MESH,
        )

        w_bf_ref[...] = w_ref[...].astype(jnp.bfloat16)

        rdmas = []
        for c in range(NCHUNK):
            a = dy_ref[pl.ds(c * rows, rows), :].astype(jnp.bfloat16)
            p = lax.dot_general(
                a, w_bf_ref[...], (((1,), (1,)), ((), ())),
                preferred_element_type=jnp.float32,
            )
            p_ref[c] = p
            q = jnp.clip(jnp.round(p * (1.0 / QSCALE)), -127.0, 127.0)
            send_buf[c] = q.astype(jnp.int8)
            if c == 0:
                pl.semaphore_wait(barrier, 1)
            rdma = pltpu.make_async_remote_copy(
                src_ref=send_buf.at[c],
                dst_ref=recv_buf.at[c],
                send_sem=send_sems.at[c],
                recv_sem=recv_sems.at[c],
                device_id=nbr,
                device_id_type=pl.DeviceIdType.MESH,
            )
            rdma.start()
            rdmas.append(rdma)

        for c in range(NCHUNK):
            rdmas[c].wait_recv()
            out_ref[pl.ds(c * rows, rows), :] = (
                p_ref[c] + recv_buf[c].astype(jnp.float32) * QSCALE
            )

        for c in range(NCHUNK):
            rdmas[c].wait_send()

    return pl.pallas_call(
        body,
        out_shape=jax.ShapeDtypeStruct((m, d), jnp.float32),
        in_specs=[
            pl.BlockSpec(memory_space=pltpu.VMEM),
            pl.BlockSpec(memory_space=pltpu.VMEM),
        ],
        out_specs=pl.BlockSpec(memory_space=pltpu.VMEM),
        scratch_shapes=[
            pltpu.VMEM((d, k), jnp.bfloat16),
            pltpu.VMEM((NCHUNK, rows, d), jnp.float32),
            pltpu.VMEM((NCHUNK, rows, d), jnp.int8),
            pltpu.VMEM((NCHUNK, rows, d), jnp.int8),
            pltpu.SemaphoreType.DMA((NCHUNK,)),
            pltpu.SemaphoreType.DMA((NCHUNK,)),
        ],
        compiler_params=pltpu.CompilerParams(collective_id=0),
    )(dy, W)
